# Initial kernel scaffold; baseline (speedup 1.0000x reference)
import functools

import jax
import jax.numpy as jnp
from jax import lax
from jax.experimental import pallas as pl
from jax.experimental.pallas import tpu as pltpu
from jax.experimental.pallas import tpu_sc as plsc

N = 10000
NP = 10240
D = 128
R = 8
E = 320000
EP = 327680
NT = 32
PER_TILE = EP // NT
CH = 128
NCH = PER_TILE // CH
BN = 1024
NB = NP // BN


def _lrelu(v):
    return jnp.where(v >= 0, v, 0.2 * v)


def _k1_body(x_ref, w_ref, qT_ref, kT_ref,
             xw_ref, sq_ref, sk_ref, gq_ref, kv_ref):
    nb = pl.program_id(0)
    r = pl.program_id(1)
    xb = x_ref[...]
    wb = w_ref[0]
    xwb = jnp.dot(xb, wb, preferred_element_type=jnp.float32)
    xw_ref[0] = xwb
    dn = (((1,), (1,)), ((), ()))
    sqb = lax.dot_general(qT_ref[...], xwb, dn,
                          preferred_element_type=jnp.float32)
    skb = lax.dot_general(kT_ref[...], xwb, dn,
                          preferred_element_type=jnp.float32)
    sq_ref[...] = sqb
    sk_ref[...] = skb

    @pl.when(r == 0)
    def _():
        gq_ref[...] = sqb

    @pl.when(r != 0)
    def _():
        gq_ref[...] = jnp.maximum(gq_ref[...], sqb)

    m = jnp.max(skb)

    @pl.when(jnp.logical_and(nb == 0, r == 0))
    def _():
        kv_ref[...] = jnp.full((1, 128), m, jnp.float32)

    @pl.when(jnp.logical_or(nb != 0, r != 0))
    def _():
        kv_ref[...] = jnp.maximum(kv_ref[...], m)


def _k1(h, w, qT, kT):
    return pl.pallas_call(
        _k1_body,
        grid=(NB, R),
        in_specs=[
            pl.BlockSpec((BN, D), lambda nb, r: (nb, 0)),
            pl.BlockSpec((1, D, D), lambda nb, r: (r, 0, 0)),
            pl.BlockSpec((1, D), lambda nb, r: (0, 0)),
            pl.BlockSpec((1, D), lambda nb, r: (0, 0)),
        ],
        out_specs=[
            pl.BlockSpec((1, BN, D), lambda nb, r: (r, nb, 0)),
            pl.BlockSpec((1, BN), lambda nb, r: (r, nb)),
            pl.BlockSpec((1, BN), lambda nb, r: (r, nb)),
            pl.BlockSpec((1, BN), lambda nb, r: (0, nb)),
            pl.BlockSpec((1, 128), lambda nb, r: (0, 0)),
        ],
        out_shape=[
            jax.ShapeDtypeStruct((R, NP, D), jnp.float32),
            jax.ShapeDtypeStruct((R, NP), jnp.float32),
            jax.ShapeDtypeStruct((R, NP), jnp.float32),
            jax.ShapeDtypeStruct((1, NP), jnp.float32),
            jax.ShapeDtypeStruct((1, 128), jnp.float32),
        ],
    )(h, w, qT, kT)


def _p1_body(src_ref, dst_ref, et_ref, sqf_ref, skf_ref, gqf_ref, kvf_ref,
             expv_ref, dpart_ref,
             sq_t, gq_t, kv_t, den_t, srcb, dstb, etb, cik, bb, eb, sem):
    cid = lax.axis_index("c")
    sid = lax.axis_index("s")
    wid = cid * 16 + sid

    pltpu.sync_copy(sqf_ref, sq_t)
    pltpu.sync_copy(gqf_ref, gq_t)
    pltpu.sync_copy(kvf_ref.at[pl.ds(0, 16)], kv_t)

    @pl.loop(0, NP // 16)
    def _zero(i):
        den_t[pl.ds(i * 16, 16)] = jnp.zeros((16,), jnp.float32)

    t0 = wid * PER_TILE

    @pl.loop(0, NCH)
    def _chunk(c):
        base = t0 + c * CH
        pltpu.sync_copy(src_ref.at[pl.ds(base, CH)], srcb)
        pltpu.sync_copy(dst_ref.at[pl.ds(base, CH)], dstb)
        pltpu.sync_copy(et_ref.at[pl.ds(base, CH)], etb)

        @pl.loop(0, CH // 16)
        def _mkidx(i):
            iv = pl.ds(i * 16, 16)
            cik[iv] = etb[iv] * NP + srcb[iv]

        pltpu.async_copy(skf_ref.at[cik], bb, sem).wait()

        @pl.loop(0, CH // 16)
        def _compute(i):
            iv = pl.ds(i * 16, 16)
            et = etb[iv]
            dv = dstb[iv]
            a = plsc.load_gather(sq_t, [et * NP + dv])
            g = plsc.load_gather(gq_t, [dv])
            b = bb[iv]
            alpha = _lrelu(a + b)
            m = _lrelu(g + kv_t[...])
            e = jnp.exp(alpha - m)
            eb[iv] = e
            plsc.addupdate_scatter(den_t, [dv], e)

        pltpu.sync_copy(eb, expv_ref.at[pl.ds(base, CH)])

    pltpu.sync_copy(den_t, dpart_ref.at[wid])


def _p1(srcp, dstp, etp, sqf, skf, gqf, kvf, mesh):
    f = functools.partial(
        pl.kernel,
        out_type=[
            jax.ShapeDtypeStruct((EP,), jnp.float32),
            jax.ShapeDtypeStruct((NT, NP), jnp.float32),
        ],
        mesh=mesh,
        scratch_types=[
            pltpu.VMEM((R * NP,), jnp.float32),
            pltpu.VMEM((NP,), jnp.float32),
            pltpu.VMEM((16,), jnp.float32),
            pltpu.VMEM((NP,), jnp.float32),
            pltpu.VMEM((CH,), jnp.int32),
            pltpu.VMEM((CH,), jnp.int32),
            pltpu.VMEM((CH,), jnp.int32),
            pltpu.VMEM((CH,), jnp.int32),
            pltpu.VMEM((CH,), jnp.float32),
            pltpu.VMEM((CH,), jnp.float32),
            pltpu.SemaphoreType.DMA,
        ],
    )(_p1_body)
    return f(srcp, dstp, etp, sqf, skf, gqf, kvf)


def _k2_body(d_ref, out_ref):
    s = jnp.sum(d_ref[...], axis=0, keepdims=True)
    out_ref[...] = 1.0 / jnp.maximum(s, 1e-30)


def _k2(dpart):
    return pl.pallas_call(
        _k2_body,
        grid=(NB,),
        in_specs=[pl.BlockSpec((NT, BN), lambda i: (0, i))],
        out_specs=pl.BlockSpec((1, BN), lambda i: (0, i)),
        out_shape=jax.ShapeDtypeStruct((1, NP), jnp.float32),
    )(dpart)


def _p2_body(src_ref, dst_ref, et_ref, expv_ref, rden_ref, xw2_ref,
             agg_ref,
             rden_t, srcb, dstb, etb, cik, wb, eb, rows, aggr, sem):
    cid = lax.axis_index("c")
    sid = lax.axis_index("s")
    wid = cid * 16 + sid
    rows_per_sub = NP // 16

    pltpu.sync_copy(rden_ref, rden_t)

    @pl.loop(0, CH)
    def _zrows(q):
        for j in range(D // 16):
            rows[q, pl.ds(j * 16, 16)] = jnp.zeros((16,), jnp.float32)

    lo = sid * rows_per_sub

    @pl.loop(0, rows_per_sub // CH)
    def _zagg(k):
        pltpu.sync_copy(rows, aggr.at[pl.ds(lo + k * CH, CH)])

    plsc.subcore_barrier()

    t0 = wid * PER_TILE

    @pl.loop(0, NCH)
    def _chunk(c):
        base = t0 + c * CH
        pltpu.sync_copy(src_ref.at[pl.ds(base, CH)], srcb)
        pltpu.sync_copy(dst_ref.at[pl.ds(base, CH)], dstb)
        pltpu.sync_copy(et_ref.at[pl.ds(base, CH)], etb)
        pltpu.sync_copy(expv_ref.at[pl.ds(base, CH)], eb)

        @pl.loop(0, CH // 16)
        def _mkidx(i):
            iv = pl.ds(i * 16, 16)
            cik[iv] = etb[iv] * NP + srcb[iv]
            wb[iv] = eb[iv] * plsc.load_gather(rden_t, [dstb[iv]])

        pltpu.async_copy(xw2_ref.at[cik], rows, sem).wait()

        @pl.loop(0, CH)
        def _scale(e):
            ws = plsc.load_gather(wb, [jnp.full((16,), 0, jnp.int32) + e])
            for j in range(D // 16):
                jv = pl.ds(j * 16, 16)
                rows[e, jv] = rows[e, jv] * ws

        pltpu.sync_copy(rows, aggr.at[dstb], add=True)

    plsc.subcore_barrier()
    pltpu.sync_copy(aggr.at[pl.ds(lo, rows_per_sub)],
                    agg_ref.at[cid, pl.ds(lo, rows_per_sub)])


def _p2(srcp, dstp, etp, expv, rden, xw2, mesh):
    f = functools.partial(
        pl.kernel,
        out_type=jax.ShapeDtypeStruct((2, NP, D), jnp.float32),
        mesh=mesh,
        scratch_types=[
            pltpu.VMEM((NP,), jnp.float32),
            pltpu.VMEM((CH,), jnp.int32),
            pltpu.VMEM((CH,), jnp.int32),
            pltpu.VMEM((CH,), jnp.int32),
            pltpu.VMEM((CH,), jnp.int32),
            pltpu.VMEM((CH,), jnp.float32),
            pltpu.VMEM((CH,), jnp.float32),
            pltpu.VMEM((CH, D), jnp.float32),
            pltpu.VMEM_SHARED((NP, D), jnp.float32),
            pltpu.SemaphoreType.DMA,
        ],
    )(_p2_body)
    return f(srcp, dstp, etp, expv, rden, xw2)


def _k4_body(a_ref, b_ref, out_ref):
    h = a_ref[0] + a_ref[1] + b_ref[...]
    out_ref[...] = jnp.maximum(h, 0.0)


def _k4(agg, b):
    return pl.pallas_call(
        _k4_body,
        grid=(NB,),
        in_specs=[
            pl.BlockSpec((2, BN, D), lambda i: (0, i, 0)),
            pl.BlockSpec((1, D), lambda i: (0, 0)),
        ],
        out_specs=pl.BlockSpec((BN, D), lambda i: (i, 0)),
        out_shape=jax.ShapeDtypeStruct((NP, D), jnp.float32),
    )(agg, b.reshape(1, D))


def _layer(h, srcp, dstp, etp, w, q, k, b, mesh):
    qT = q.reshape(1, D)
    kT = k.reshape(1, D)
    xw, sq, sk, gq, kv = _k1(h, w, qT, kT)
    expv, dpart = _p1(srcp, dstp, etp,
                      sq.reshape(R * NP), sk.reshape(R * NP),
                      gq.reshape(NP), kv.reshape(128), mesh)
    rden = _k2(dpart)
    agg = _p2(srcp, dstp, etp, expv, rden.reshape(NP),
              xw.reshape(R * NP, D), mesh)
    return _k4(agg, b)


def kernel(x, edge_index, edge_type, w0, q0, k0, b0, w1, q1, k1, b1):
    mesh = plsc.VectorSubcoreMesh(core_axis_name="c", subcore_axis_name="s")
    src = edge_index[0]
    dst = edge_index[1]
    pad = jnp.full((EP - E,), NP - 1, jnp.int32)
    srcp = jnp.concatenate([src, pad])
    dstp = jnp.concatenate([dst, pad])
    etp = jnp.concatenate([edge_type, jnp.zeros((EP - E,), jnp.int32)])
    xp = jnp.pad(x, ((0, NP - N), (0, 0)))
    h = _layer(xp, srcp, dstp, etp, w0, q0, k0, b0, mesh)
    h = _layer(h, srcp, dstp, etp, w1, q1, k1, b1, mesh)
    return h[:N]


# speedup vs baseline: 11.5434x; 11.5434x over previous
import functools

import jax
import jax.numpy as jnp
from jax import lax
from jax.experimental import pallas as pl
from jax.experimental.pallas import tpu as pltpu
from jax.experimental.pallas import tpu_sc as plsc

N = 10000
NP = 10240
D = 128
R = 8
E = 320000
EP = 327680
NT = 32
PER_TILE = EP // NT
CH = 128
NCH = PER_TILE // CH
BN = 1024
NB = NP // BN
RPS = NP // 16

f32 = jnp.float32
i32 = jnp.int32


def _lrelu(v):
    return jnp.where(v >= 0, v, 0.2 * v)


def _k1_body(x_ref, w_ref, qT_ref, kT_ref,
             xw_ref, sq_ref, sk_ref, gq_ref, kv_ref):
    nb = pl.program_id(0)
    r = pl.program_id(1)
    xb = x_ref[...]
    wb = w_ref[0]
    xwb = jnp.dot(xb, wb, preferred_element_type=f32)
    xw_ref[0] = xwb
    dn = (((1,), (1,)), ((), ()))
    sqb = lax.dot_general(qT_ref[...], xwb, dn, preferred_element_type=f32)
    skb = lax.dot_general(kT_ref[...], xwb, dn, preferred_element_type=f32)
    sq_ref[0] = sqb
    sk_ref[0] = skb

    @pl.when(r == 0)
    def _():
        gq_ref[...] = sqb

    @pl.when(r != 0)
    def _():
        gq_ref[...] = jnp.maximum(gq_ref[...], sqb)

    m = jnp.max(skb)

    @pl.when(jnp.logical_and(nb == 0, r == 0))
    def _():
        kv_ref[...] = jnp.full((1, 128), m, f32)

    @pl.when(jnp.logical_or(nb != 0, r != 0))
    def _():
        kv_ref[...] = jnp.maximum(kv_ref[...], m)


def _k1(h, w, qT, kT):
    return pl.pallas_call(
        _k1_body,
        grid=(NB, R),
        in_specs=[
            pl.BlockSpec((BN, D), lambda nb, r: (nb, 0)),
            pl.BlockSpec((1, D, D), lambda nb, r: (r, 0, 0)),
            pl.BlockSpec((1, D), lambda nb, r: (0, 0)),
            pl.BlockSpec((1, D), lambda nb, r: (0, 0)),
        ],
        out_specs=[
            pl.BlockSpec((1, BN, D), lambda nb, r: (r, nb, 0)),
            pl.BlockSpec((1, 1, BN), lambda nb, r: (r, 0, nb)),
            pl.BlockSpec((1, 1, BN), lambda nb, r: (r, 0, nb)),
            pl.BlockSpec((1, BN), lambda nb, r: (0, nb)),
            pl.BlockSpec((1, 128), lambda nb, r: (0, 0)),
        ],
        out_shape=[
            jax.ShapeDtypeStruct((R, NP, D), f32),
            jax.ShapeDtypeStruct((R, 1, NP), f32),
            jax.ShapeDtypeStruct((R, 1, NP), f32),
            jax.ShapeDtypeStruct((1, NP), f32),
            jax.ShapeDtypeStruct((1, 128), f32),
        ],
    )(h, w, qT, kT)


def _p1_body(src_ref, dst_ref, et_ref, sqf_ref, skf_ref, gqf_ref, kvf_ref,
             expv_ref, dpart_ref,
             kv_t, zb, srcb, dstb, etb, cik, ciq, ab, bb, gb, eb, den, sem):
    cid = lax.axis_index("c")
    sid = lax.axis_index("s")
    wid = cid * 16 + sid

    pltpu.sync_copy(kvf_ref.at[pl.ds(0, 16)], kv_t)

    @pl.loop(0, RPS // 16)
    def _z(i):
        zb[pl.ds(i * 16, 16)] = jnp.zeros((16,), f32)

    pltpu.sync_copy(zb, den.at[pl.ds(sid * RPS, RPS)])
    plsc.subcore_barrier()

    t0 = wid * PER_TILE

    @pl.loop(0, NCH)
    def _chunk(c):
        base = t0 + c * CH
        pltpu.sync_copy(src_ref.at[pl.ds(base, CH)], srcb)
        pltpu.sync_copy(dst_ref.at[pl.ds(base, CH)], dstb)
        pltpu.sync_copy(et_ref.at[pl.ds(base, CH)], etb)

        @pl.loop(0, CH // 16)
        def _mkidx(i):
            iv = pl.ds(i * 16, 16)
            et = etb[iv]
            cik[iv] = et * NP + srcb[iv]
            ciq[iv] = et * NP + dstb[iv]

        pltpu.async_copy(skf_ref.at[cik], bb, sem).wait()
        pltpu.async_copy(sqf_ref.at[ciq], ab, sem).wait()
        pltpu.async_copy(gqf_ref.at[dstb], gb, sem).wait()

        kvv = kv_t[...]

        @pl.loop(0, CH // 16)
        def _compute(i):
            iv = pl.ds(i * 16, 16)
            alpha = _lrelu(ab[iv] + bb[iv])
            m = _lrelu(gb[iv] + kvv)
            eb[iv] = jnp.exp(alpha - m)

        pltpu.sync_copy(eb, expv_ref.at[pl.ds(base, CH)])
        pltpu.sync_copy(eb, den.at[dstb], add=True)

    plsc.subcore_barrier()

    @pl.when(sid == 0)
    def _():
        pltpu.sync_copy(den, dpart_ref.at[cid])


def _p1(srcp, dstp, etp, sqf, skf, gqf, kvf, mesh):
    f = functools.partial(
        pl.kernel,
        out_type=[
            jax.ShapeDtypeStruct((EP,), f32),
            jax.ShapeDtypeStruct((2, NP), f32),
        ],
        mesh=mesh,
        scratch_types=[
            pltpu.VMEM((16,), f32),
            pltpu.VMEM((RPS,), f32),
            pltpu.VMEM((CH,), i32),
            pltpu.VMEM((CH,), i32),
            pltpu.VMEM((CH,), i32),
            pltpu.VMEM((CH,), i32),
            pltpu.VMEM((CH,), i32),
            pltpu.VMEM((CH,), f32),
            pltpu.VMEM((CH,), f32),
            pltpu.VMEM((CH,), f32),
            pltpu.VMEM((CH,), f32),
            pltpu.VMEM_SHARED((NP,), f32),
            pltpu.SemaphoreType.DMA,
        ],
    )(_p1_body)
    return f(srcp, dstp, etp, sqf, skf, gqf, kvf)


def _k2_body(d_ref, out_ref):
    s = jnp.sum(d_ref[...], axis=0, keepdims=True)
    out_ref[...] = 1.0 / jnp.maximum(s, 1e-30)


def _k2(dpart):
    return pl.pallas_call(
        _k2_body,
        grid=(NB,),
        in_specs=[pl.BlockSpec((2, BN), lambda i: (0, i))],
        out_specs=pl.BlockSpec((1, BN), lambda i: (0, i)),
        out_shape=jax.ShapeDtypeStruct((1, NP), f32),
    )(dpart)


def _p2_body(src_ref, dst_ref, et_ref, expv_ref, rden_ref, xw2_ref,
             agg_ref,
             srcb, dstb, etb, cik, wb, eb, rb, rows, aggr, sem):
    cid = lax.axis_index("c")
    sid = lax.axis_index("s")
    wid = cid * 16 + sid

    @pl.loop(0, CH)
    def _zrows(q):
        for j in range(D // 16):
            rows[q, pl.ds(j * 16, 16)] = jnp.zeros((16,), f32)

    lo = sid * RPS

    @pl.loop(0, RPS // CH)
    def _zagg(k):
        pltpu.sync_copy(rows, aggr.at[pl.ds(lo + k * CH, CH)])

    plsc.subcore_barrier()

    t0 = wid * PER_TILE

    @pl.loop(0, NCH)
    def _chunk(c):
        base = t0 + c * CH
        pltpu.sync_copy(src_ref.at[pl.ds(base, CH)], srcb)
        pltpu.sync_copy(dst_ref.at[pl.ds(base, CH)], dstb)
        pltpu.sync_copy(et_ref.at[pl.ds(base, CH)], etb)
        pltpu.sync_copy(expv_ref.at[pl.ds(base, CH)], eb)

        @pl.loop(0, CH // 16)
        def _mkidx(i):
            iv = pl.ds(i * 16, 16)
            cik[iv] = etb[iv] * NP + srcb[iv]

        pltpu.async_copy(rden_ref.at[dstb], rb, sem).wait()

        @pl.loop(0, CH // 16)
        def _mkw(i):
            iv = pl.ds(i * 16, 16)
            wb[iv] = eb[iv] * rb[iv]

        pltpu.async_copy(xw2_ref.at[cik], rows, sem).wait()

        @pl.loop(0, CH // 16)
        def _scale(g):
            wv = wb[pl.ds(g * 16, 16)]
            for l in range(16):
                ws = jnp.full((16,), wv[l], f32)
                e = g * 16 + l
                for j in range(D // 16):
                    jv = pl.ds(j * 16, 16)
                    rows[e, jv] = rows[e, jv] * ws

        pltpu.sync_copy(rows, aggr.at[dstb], add=True)

    plsc.subcore_barrier()
    pltpu.sync_copy(aggr.at[pl.ds(lo, RPS)],
                    agg_ref.at[cid, pl.ds(lo, RPS)])


def _p2(srcp, dstp, etp, expv, rden, xw2, mesh):
    f = functools.partial(
        pl.kernel,
        out_type=jax.ShapeDtypeStruct((2, NP, D), f32),
        mesh=mesh,
        scratch_types=[
            pltpu.VMEM((CH,), i32),
            pltpu.VMEM((CH,), i32),
            pltpu.VMEM((CH,), i32),
            pltpu.VMEM((CH,), i32),
            pltpu.VMEM((CH,), f32),
            pltpu.VMEM((CH,), f32),
            pltpu.VMEM((CH,), f32),
            pltpu.VMEM((CH, D), f32),
            pltpu.VMEM_SHARED((NP, D), f32),
            pltpu.SemaphoreType.DMA,
        ],
    )(_p2_body)
    return f(srcp, dstp, etp, expv, rden, xw2)


def _k4_body(a_ref, b_ref, out_ref):
    h = a_ref[0] + a_ref[1] + b_ref[...]
    out_ref[...] = jnp.maximum(h, 0.0)


def _k4(agg, b):
    return pl.pallas_call(
        _k4_body,
        grid=(NB,),
        in_specs=[
            pl.BlockSpec((2, BN, D), lambda i: (0, i, 0)),
            pl.BlockSpec((1, D), lambda i: (0, 0)),
        ],
        out_specs=pl.BlockSpec((BN, D), lambda i: (i, 0)),
        out_shape=jax.ShapeDtypeStruct((NP, D), f32),
    )(agg, b.reshape(1, D))


def _layer(h, srcp, dstp, etp, w, q, k, b, mesh):
    qT = q.reshape(1, D)
    kT = k.reshape(1, D)
    xw, sq, sk, gq, kv = _k1(h, w, qT, kT)
    expv, dpart = _p1(srcp, dstp, etp,
                      sq.reshape(R * NP), sk.reshape(R * NP),
                      gq.reshape(NP), kv.reshape(128), mesh)
    rden = _k2(dpart)
    agg = _p2(srcp, dstp, etp, expv, rden.reshape(NP),
              xw.reshape(R * NP, D), mesh)
    return _k4(agg, b)


def kernel(x, edge_index, edge_type, w0, q0, k0, b0, w1, q1, k1, b1):
    mesh = plsc.VectorSubcoreMesh(core_axis_name="c", subcore_axis_name="s")
    src = edge_index[0]
    dst = edge_index[1]
    pad = jnp.full((EP - E,), NP - 1, i32)
    srcp = jnp.concatenate([src, pad])
    dstp = jnp.concatenate([dst, pad])
    etp = jnp.concatenate([edge_type, jnp.zeros((EP - E,), i32)])
    xp = jnp.pad(x, ((0, NP - N), (0, 0)))
    h = _layer(xp, srcp, dstp, etp, w0, q0, k0, b0, mesh)
    h = _layer(h, srcp, dstp, etp, w1, q1, k1, b1, mesh)
    return h[:N]


# speedup vs baseline: 17.0058x; 1.4732x over previous
import functools

import jax
import jax.numpy as jnp
from jax import lax
from jax.experimental import pallas as pl
from jax.experimental.pallas import tpu as pltpu
from jax.experimental.pallas import tpu_sc as plsc

N = 10000
NP = 10240
D = 128
R = 8
E = 320000
EP = 327680
NT = 32
PER_TILE = EP // NT
CH = 128
NCH = PER_TILE // CH
BN = 1024
NB = NP // BN
RPS = NP // 16

f32 = jnp.float32
i32 = jnp.int32


def _lrelu(v):
    return jnp.where(v >= 0, v, 0.2 * v)


def _k1_body(x_ref, w_ref, qT_ref, kT_ref,
             xw_ref, sq_ref, sk_ref, gq_ref, kv_ref):
    nb = pl.program_id(0)
    r = pl.program_id(1)
    xb = x_ref[...]
    wb = w_ref[0]
    xwb = jnp.dot(xb, wb, preferred_element_type=f32)
    xw_ref[0] = xwb
    dn = (((1,), (1,)), ((), ()))
    sqb = lax.dot_general(qT_ref[...], xwb, dn, preferred_element_type=f32)
    skb = lax.dot_general(kT_ref[...], xwb, dn, preferred_element_type=f32)
    sq_ref[0] = sqb
    sk_ref[0] = skb

    @pl.when(r == 0)
    def _():
        gq_ref[...] = sqb

    @pl.when(r != 0)
    def _():
        gq_ref[...] = jnp.maximum(gq_ref[...], sqb)

    m = jnp.max(skb)

    @pl.when(jnp.logical_and(nb == 0, r == 0))
    def _():
        kv_ref[...] = jnp.full((1, 128), m, f32)

    @pl.when(jnp.logical_or(nb != 0, r != 0))
    def _():
        kv_ref[...] = jnp.maximum(kv_ref[...], m)


def _k1(h, w, qT, kT):
    return pl.pallas_call(
        _k1_body,
        grid=(NB, R),
        in_specs=[
            pl.BlockSpec((BN, D), lambda nb, r: (nb, 0)),
            pl.BlockSpec((1, D, D), lambda nb, r: (r, 0, 0)),
            pl.BlockSpec((1, D), lambda nb, r: (0, 0)),
            pl.BlockSpec((1, D), lambda nb, r: (0, 0)),
        ],
        out_specs=[
            pl.BlockSpec((1, BN, D), lambda nb, r: (r, nb, 0)),
            pl.BlockSpec((1, 1, BN), lambda nb, r: (r, 0, nb)),
            pl.BlockSpec((1, 1, BN), lambda nb, r: (r, 0, nb)),
            pl.BlockSpec((1, BN), lambda nb, r: (0, nb)),
            pl.BlockSpec((1, 128), lambda nb, r: (0, 0)),
        ],
        out_shape=[
            jax.ShapeDtypeStruct((R, NP, D), f32),
            jax.ShapeDtypeStruct((R, 1, NP), f32),
            jax.ShapeDtypeStruct((R, 1, NP), f32),
            jax.ShapeDtypeStruct((1, NP), f32),
            jax.ShapeDtypeStruct((1, 128), f32),
        ],
    )(h, w, qT, kT)


def _p1_body(src_ref, dst_ref, et_ref, sqf_ref, skf_ref, gqf_ref, kvf_ref,
             expv_ref, dpart_ref,
             kv_t, zb, srcb, dstb, etb, cik, ciq, ab, bb, gb, eb, den, sem):
    cid = lax.axis_index("c")
    sid = lax.axis_index("s")
    wid = cid * 16 + sid

    pltpu.sync_copy(kvf_ref.at[pl.ds(0, 16)], kv_t)

    @pl.loop(0, RPS // 16)
    def _z(i):
        zb[pl.ds(i * 16, 16)] = jnp.zeros((16,), f32)

    pltpu.sync_copy(zb, den.at[pl.ds(sid * RPS, RPS)])
    plsc.subcore_barrier()

    t0 = wid * PER_TILE

    @pl.loop(0, NCH)
    def _chunk(c):
        base = t0 + c * CH
        pltpu.sync_copy(src_ref.at[pl.ds(base, CH)], srcb)
        pltpu.sync_copy(dst_ref.at[pl.ds(base, CH)], dstb)
        pltpu.sync_copy(et_ref.at[pl.ds(base, CH)], etb)

        @pl.loop(0, CH // 16)
        def _mkidx(i):
            iv = pl.ds(i * 16, 16)
            et = etb[iv]
            cik[iv] = et * NP + srcb[iv]
            ciq[iv] = et * NP + dstb[iv]

        pltpu.async_copy(skf_ref.at[cik], bb, sem).wait()
        pltpu.async_copy(sqf_ref.at[ciq], ab, sem).wait()
        pltpu.async_copy(gqf_ref.at[dstb], gb, sem).wait()

        kvv = kv_t[...]

        @pl.loop(0, CH // 16)
        def _compute(i):
            iv = pl.ds(i * 16, 16)
            alpha = _lrelu(ab[iv] + bb[iv])
            m = _lrelu(gb[iv] + kvv)
            eb[iv] = jnp.exp(alpha - m)

        pltpu.sync_copy(eb, expv_ref.at[pl.ds(base, CH)])
        pltpu.sync_copy(eb, den.at[dstb], add=True)

    plsc.subcore_barrier()

    @pl.when(sid == 0)
    def _():
        pltpu.sync_copy(den, dpart_ref.at[cid])


def _p1(srcp, dstp, etp, sqf, skf, gqf, kvf, mesh):
    f = functools.partial(
        pl.kernel,
        out_type=[
            jax.ShapeDtypeStruct((EP,), f32),
            jax.ShapeDtypeStruct((2, NP), f32),
        ],
        mesh=mesh,
        scratch_types=[
            pltpu.VMEM((16,), f32),
            pltpu.VMEM((RPS,), f32),
            pltpu.VMEM((CH,), i32),
            pltpu.VMEM((CH,), i32),
            pltpu.VMEM((CH,), i32),
            pltpu.VMEM((CH,), i32),
            pltpu.VMEM((CH,), i32),
            pltpu.VMEM((CH,), f32),
            pltpu.VMEM((CH,), f32),
            pltpu.VMEM((CH,), f32),
            pltpu.VMEM((CH,), f32),
            pltpu.VMEM_SHARED((NP,), f32),
            pltpu.SemaphoreType.DMA,
        ],
    )(_p1_body)
    return f(srcp, dstp, etp, sqf, skf, gqf, kvf)


BC = 8
NBC = NCH // BC


def _p2_body(meta_ref, expv_ref, xw2_ref,
             agg_ref,
             mbig0, mbig1, ebig, cik8, dst8, rows0, rows1, aggr,
             semm0, semm1, semg0, semg1):
    cid = lax.axis_index("c")
    sid = lax.axis_index("s")
    wid = cid * 16 + sid
    mbig = [mbig0, mbig1]
    rows = [rows0, rows1]
    semm = [semm0, semm1]
    semg = [semg0, semg1]

    @pl.loop(0, CH)
    def _zrows(q):
        for j in range(D // 16):
            rows0[q, pl.ds(j * 16, 16)] = jnp.zeros((16,), f32)

    lo = sid * RPS

    @pl.loop(0, RPS // CH)
    def _zagg(k):
        pltpu.sync_copy(rows0, aggr.at[pl.ds(lo + k * CH, CH)])

    plsc.subcore_barrier()

    t0 = wid * PER_TILE
    t0c = wid * NCH

    pltpu.async_copy(meta_ref.at[pl.ds(t0c, BC)], mbig0, semm0)

    @pl.loop(0, NBC // 2)
    def _blocks(tt):
        for pb in range(2):
            bb = 2 * tt + pb
            pltpu.make_async_copy(meta_ref.at[pl.ds(t0c + bb * BC, BC)],
                                  mbig[pb], semm[pb]).wait()

            @pl.when(bb + 1 < NBC)
            def _():
                pltpu.async_copy(
                    meta_ref.at[pl.ds(t0c + (bb + 1) * BC, BC)],
                    mbig[1 - pb], semm[1 - pb])

            pltpu.sync_copy(expv_ref.at[pl.ds(t0 + bb * BC * CH, BC * CH)],
                            ebig)

            @pl.loop(0, BC)
            def _mk(j):
                @pl.loop(0, CH // 16)
                def _mki(i):
                    iv = pl.ds(i * 16, 16)
                    dst8[j, 0, iv] = mbig[pb][j, 1, iv]
                    cik8[j, 0, iv] = (mbig[pb][j, 2, iv] * NP
                                      + mbig[pb][j, 0, iv])

            pltpu.async_copy(xw2_ref.at[cik8.at[0, 0]], rows0, semg0)

            @pl.loop(0, BC // 2)
            def _pipe(p):
                for b in range(2):
                    cc = 2 * p + b
                    pltpu.make_async_copy(xw2_ref.at[cik8.at[cc, 0]],
                                          rows[b], semg[b]).wait()

                    @pl.when(cc + 1 < BC)
                    def _():
                        pltpu.async_copy(xw2_ref.at[cik8.at[cc + 1, 0]],
                                         rows[1 - b], semg[1 - b])

                    @pl.loop(0, CH // 16)
                    def _scale(g):
                        wv = ebig[pl.ds(cc * CH + g * 16, 16)]
                        for l in range(16):
                            ws = jnp.full((16,), wv[l], f32)
                            e = g * 16 + l
                            for j in range(D // 16):
                                jv = pl.ds(j * 16, 16)
                                rows[b][e, jv] = rows[b][e, jv] * ws

                    pltpu.sync_copy(rows[b], aggr.at[dst8.at[cc, 0]],
                                    add=True)

    plsc.subcore_barrier()
    pltpu.sync_copy(aggr.at[pl.ds(lo, RPS)],
                    agg_ref.at[cid, pl.ds(lo, RPS)])


def _p2(meta3, expv, xw2, mesh):
    f = functools.partial(
        pl.kernel,
        out_type=jax.ShapeDtypeStruct((2, NP, D), f32),
        mesh=mesh,
        scratch_types=[
            pltpu.VMEM((BC, 3, CH), i32),
            pltpu.VMEM((BC, 3, CH), i32),
            pltpu.VMEM((BC * CH,), f32),
            pltpu.VMEM((BC, 1, CH), i32),
            pltpu.VMEM((BC, 1, CH), i32),
            pltpu.VMEM((CH, D), f32),
            pltpu.VMEM((CH, D), f32),
            pltpu.VMEM_SHARED((NP, D), f32),
            pltpu.SemaphoreType.DMA,
            pltpu.SemaphoreType.DMA,
            pltpu.SemaphoreType.DMA,
            pltpu.SemaphoreType.DMA,
        ],
    )(_p2_body)
    return f(meta3, expv, xw2)


def _k4_body(a_ref, d_ref, b_ref, out_ref):
    ones = jnp.ones((2, 1), f32)
    s = lax.dot_general(d_ref[...], ones, (((0,), (0,)), ((), ())),
                        preferred_element_type=f32)
    rden = 1.0 / jnp.maximum(s, 1e-30)
    h = (a_ref[0] + a_ref[1]) * rden + b_ref[...]
    out_ref[...] = jnp.maximum(h, 0.0)


def _k4(agg, dpart, b):
    return pl.pallas_call(
        _k4_body,
        grid=(NB,),
        in_specs=[
            pl.BlockSpec((2, BN, D), lambda i: (0, i, 0)),
            pl.BlockSpec((2, BN), lambda i: (0, i)),
            pl.BlockSpec((1, D), lambda i: (0, 0)),
        ],
        out_specs=pl.BlockSpec((BN, D), lambda i: (i, 0)),
        out_shape=jax.ShapeDtypeStruct((NP, D), f32),
    )(agg, dpart, b.reshape(1, D))


def _layer(h, srcp, dstp, etp, meta3, w, q, k, b, mesh):
    qT = q.reshape(1, D)
    kT = k.reshape(1, D)
    xw, sq, sk, gq, kv = _k1(h, w, qT, kT)
    expv, dpart = _p1(srcp, dstp, etp,
                      sq.reshape(R * NP), sk.reshape(R * NP),
                      gq.reshape(NP), kv.reshape(128), mesh)
    agg = _p2(meta3, expv, xw.reshape(R * NP, D), mesh)
    return _k4(agg, dpart, b)


def kernel(x, edge_index, edge_type, w0, q0, k0, b0, w1, q1, k1, b1):
    mesh = plsc.VectorSubcoreMesh(core_axis_name="c", subcore_axis_name="s")
    src = edge_index[0]
    dst = edge_index[1]
    pad = jnp.full((EP - E,), NP - 1, i32)
    srcp = jnp.concatenate([src, pad])
    dstp = jnp.concatenate([dst, pad])
    etp = jnp.concatenate([edge_type, jnp.zeros((EP - E,), i32)])
    meta3 = jnp.transpose(
        jnp.stack([srcp, dstp, etp]).reshape(3, EP // CH, CH), (1, 0, 2))
    xp = jnp.pad(x, ((0, NP - N), (0, 0)))
    h = _layer(xp, srcp, dstp, etp, meta3, w0, q0, k0, b0, mesh)
    h = _layer(h, srcp, dstp, etp, meta3, w1, q1, k1, b1, mesh)
    return h[:N]


# speedup vs baseline: 21.8131x; 1.2827x over previous
import functools

import jax
import jax.numpy as jnp
from jax import lax
from jax.experimental import pallas as pl
from jax.experimental.pallas import tpu as pltpu
from jax.experimental.pallas import tpu_sc as plsc

N = 10000
NP = 10240
D = 128
R = 8
E = 320000
EP = 327680
NT = 32
PER_TILE = EP // NT
CH = 128
NCH = PER_TILE // CH
BN = 1024
NB = NP // BN
RPS = NP // 16

f32 = jnp.float32
i32 = jnp.int32


def _lrelu(v):
    return jnp.where(v >= 0, v, 0.2 * v)


def _k1_body(x_ref, w_ref, qT_ref, kT_ref,
             xw_ref, sq_ref, sk_ref, gq_ref, kv_ref):
    nb = pl.program_id(0)
    r = pl.program_id(1)
    xb = x_ref[...]
    wb = w_ref[0]
    xwb = jnp.dot(xb, wb, preferred_element_type=f32)
    xw_ref[0] = xwb
    dn = (((1,), (1,)), ((), ()))
    sqb = lax.dot_general(qT_ref[...], xwb, dn, preferred_element_type=f32)
    skb = lax.dot_general(kT_ref[...], xwb, dn, preferred_element_type=f32)
    sq_ref[0] = sqb
    sk_ref[0] = skb

    @pl.when(r == 0)
    def _():
        gq_ref[...] = sqb

    @pl.when(r != 0)
    def _():
        gq_ref[...] = jnp.maximum(gq_ref[...], sqb)

    m = jnp.max(skb)

    @pl.when(jnp.logical_and(nb == 0, r == 0))
    def _():
        kv_ref[...] = jnp.full((1, 128), m, f32)

    @pl.when(jnp.logical_or(nb != 0, r != 0))
    def _():
        kv_ref[...] = jnp.maximum(kv_ref[...], m)


def _k1(h, w, qT, kT):
    return pl.pallas_call(
        _k1_body,
        grid=(NB, R),
        in_specs=[
            pl.BlockSpec((BN, D), lambda nb, r: (nb, 0)),
            pl.BlockSpec((1, D, D), lambda nb, r: (r, 0, 0)),
            pl.BlockSpec((1, D), lambda nb, r: (0, 0)),
            pl.BlockSpec((1, D), lambda nb, r: (0, 0)),
        ],
        out_specs=[
            pl.BlockSpec((1, BN, D), lambda nb, r: (r, nb, 0)),
            pl.BlockSpec((1, 1, BN), lambda nb, r: (r, 0, nb)),
            pl.BlockSpec((1, 1, BN), lambda nb, r: (r, 0, nb)),
            pl.BlockSpec((1, BN), lambda nb, r: (0, nb)),
            pl.BlockSpec((1, 128), lambda nb, r: (0, 0)),
        ],
        out_shape=[
            jax.ShapeDtypeStruct((R, NP, D), f32),
            jax.ShapeDtypeStruct((R, 1, NP), f32),
            jax.ShapeDtypeStruct((R, 1, NP), f32),
            jax.ShapeDtypeStruct((1, NP), f32),
            jax.ShapeDtypeStruct((1, 128), f32),
        ],
    )(h, w, qT, kT)


def _p1_body(meta_ref, sqf_ref, skf_ref, gqf_ref, kvf_ref,
             expv_ref, dpart_ref,
             kv_t, zb, mbig0, mbig1, cik8, ciq8,
             ab0, ab1, bb0, bb1, gb0, gb1, eb0, eb1, den,
             semm0, semm1, semg0, semg1, semo0, semo1, semd0, semd1):
    cid = lax.axis_index("c")
    sid = lax.axis_index("s")
    wid = cid * 16 + sid
    mbig = [mbig0, mbig1]
    ab = [ab0, ab1]
    bb = [bb0, bb1]
    gb = [gb0, gb1]
    eb = [eb0, eb1]
    semm = [semm0, semm1]
    semg = [semg0, semg1]
    semo = [semo0, semo1]
    semd = [semd0, semd1]

    pltpu.sync_copy(kvf_ref.at[pl.ds(0, 16)], kv_t)

    @pl.loop(0, RPS // 16)
    def _z(i):
        zb[pl.ds(i * 16, 16)] = jnp.zeros((16,), f32)

    pltpu.sync_copy(zb, den.at[pl.ds(sid * RPS, RPS)])
    plsc.subcore_barrier()

    t0 = wid * PER_TILE
    t0c = wid * NCH
    kvv = kv_t[...]

    def _fire_gathers(pb, cc, b):
        pltpu.async_copy(skf_ref.at[cik8.at[cc, 0]], bb[b], semg[b])
        pltpu.async_copy(sqf_ref.at[ciq8.at[cc, 0]], ab[b], semg[b])
        pltpu.async_copy(gqf_ref.at[mbig[pb].at[cc, 1]], gb[b], semg[b])

    def _wait_gathers(pb, cc, b):
        pltpu.make_async_copy(skf_ref.at[cik8.at[cc, 0]], bb[b],
                              semg[b]).wait()
        pltpu.make_async_copy(sqf_ref.at[ciq8.at[cc, 0]], ab[b],
                              semg[b]).wait()
        pltpu.make_async_copy(gqf_ref.at[mbig[pb].at[cc, 1]], gb[b],
                              semg[b]).wait()

    pltpu.async_copy(meta_ref.at[pl.ds(t0c, BC)], mbig0, semm0)

    @pl.loop(0, NBC // 2)
    def _blocks(tt):
        for pb in range(2):
            bb_ = 2 * tt + pb
            pltpu.make_async_copy(meta_ref.at[pl.ds(t0c + bb_ * BC, BC)],
                                  mbig[pb], semm[pb]).wait()

            @pl.when(bb_ + 1 < NBC)
            def _():
                pltpu.async_copy(
                    meta_ref.at[pl.ds(t0c + (bb_ + 1) * BC, BC)],
                    mbig[1 - pb], semm[1 - pb])

            @pl.loop(0, BC)
            def _mk(j):
                @pl.loop(0, CH // 16)
                def _mki(i):
                    iv = pl.ds(i * 16, 16)
                    et = mbig[pb][j, 2, iv]
                    cik8[j, 0, iv] = et * NP + mbig[pb][j, 0, iv]
                    ciq8[j, 0, iv] = et * NP + mbig[pb][j, 1, iv]

            _fire_gathers(pb, 0, 0)

            @pl.loop(0, BC // 2)
            def _pipe(p):
                for b in range(2):
                    cc = 2 * p + b
                    _wait_gathers(pb, cc, b)

                    @pl.when(cc + 1 < BC)
                    def _():
                        _fire_gathers(pb, cc + 1, 1 - b)

                    base = t0 + (bb_ * BC + cc) * CH

                    @pl.when(cc >= 2)
                    def _():
                        pltpu.make_async_copy(
                            eb[b], expv_ref.at[pl.ds(base - 2 * CH, CH)],
                            semo[b]).wait()
                        pltpu.make_async_copy(
                            eb[b], den.at[mbig[pb].at[cc - 2, 1]],
                            semd[b]).wait()

                    @pl.loop(0, CH // 16)
                    def _compute(i):
                        iv = pl.ds(i * 16, 16)
                        alpha = _lrelu(ab[b][iv] + bb[b][iv])
                        m = _lrelu(gb[b][iv] + kvv)
                        eb[b][iv] = jnp.exp(alpha - m)

                    pltpu.async_copy(eb[b], expv_ref.at[pl.ds(base, CH)],
                                     semo[b])
                    pltpu.async_copy(eb[b], den.at[mbig[pb].at[cc, 1]],
                                     semd[b], add=True)

            for b in range(2):
                cc = BC - 2 + b
                base = t0 + (bb_ * BC + cc) * CH
                pltpu.make_async_copy(eb[b], expv_ref.at[pl.ds(base, CH)],
                                      semo[b]).wait()
                pltpu.make_async_copy(eb[b], den.at[mbig[pb].at[cc, 1]],
                                      semd[b]).wait()

    plsc.subcore_barrier()

    @pl.when(sid == 0)
    def _():
        pltpu.sync_copy(den, dpart_ref.at[cid])


def _p1(meta3, sqf, skf, gqf, kvf, mesh):
    f = functools.partial(
        pl.kernel,
        out_type=[
            jax.ShapeDtypeStruct((EP,), f32),
            jax.ShapeDtypeStruct((2, NP), f32),
        ],
        mesh=mesh,
        scratch_types=[
            pltpu.VMEM((16,), f32),
            pltpu.VMEM((RPS,), f32),
            pltpu.VMEM((BC, 3, CH), i32),
            pltpu.VMEM((BC, 3, CH), i32),
            pltpu.VMEM((BC, 1, CH), i32),
            pltpu.VMEM((BC, 1, CH), i32),
            pltpu.VMEM((CH,), f32),
            pltpu.VMEM((CH,), f32),
            pltpu.VMEM((CH,), f32),
            pltpu.VMEM((CH,), f32),
            pltpu.VMEM((CH,), f32),
            pltpu.VMEM((CH,), f32),
            pltpu.VMEM((CH,), f32),
            pltpu.VMEM((CH,), f32),
            pltpu.VMEM_SHARED((NP,), f32),
            pltpu.SemaphoreType.DMA,
            pltpu.SemaphoreType.DMA,
            pltpu.SemaphoreType.DMA,
            pltpu.SemaphoreType.DMA,
            pltpu.SemaphoreType.DMA,
            pltpu.SemaphoreType.DMA,
            pltpu.SemaphoreType.DMA,
            pltpu.SemaphoreType.DMA,
        ],
    )(_p1_body)
    return f(meta3, sqf, skf, gqf, kvf)


BC = 8
NBC = NCH // BC


def _p2_body(meta_ref, expv_ref, xw2_ref,
             agg_ref,
             mbig0, mbig1, ebig, cik8, dst8, rows0, rows1, aggr,
             semm0, semm1, semg0, semg1):
    cid = lax.axis_index("c")
    sid = lax.axis_index("s")
    wid = cid * 16 + sid
    mbig = [mbig0, mbig1]
    rows = [rows0, rows1]
    semm = [semm0, semm1]
    semg = [semg0, semg1]

    @pl.loop(0, CH)
    def _zrows(q):
        for j in range(D // 16):
            rows0[q, pl.ds(j * 16, 16)] = jnp.zeros((16,), f32)

    lo = sid * RPS

    @pl.loop(0, RPS // CH)
    def _zagg(k):
        pltpu.sync_copy(rows0, aggr.at[pl.ds(lo + k * CH, CH)])

    plsc.subcore_barrier()

    t0 = wid * PER_TILE
    t0c = wid * NCH

    pltpu.async_copy(meta_ref.at[pl.ds(t0c, BC)], mbig0, semm0)

    @pl.loop(0, NBC // 2)
    def _blocks(tt):
        for pb in range(2):
            bb = 2 * tt + pb
            pltpu.make_async_copy(meta_ref.at[pl.ds(t0c + bb * BC, BC)],
                                  mbig[pb], semm[pb]).wait()

            @pl.when(bb + 1 < NBC)
            def _():
                pltpu.async_copy(
                    meta_ref.at[pl.ds(t0c + (bb + 1) * BC, BC)],
                    mbig[1 - pb], semm[1 - pb])

            pltpu.sync_copy(expv_ref.at[pl.ds(t0 + bb * BC * CH, BC * CH)],
                            ebig)

            @pl.loop(0, BC)
            def _mk(j):
                @pl.loop(0, CH // 16)
                def _mki(i):
                    iv = pl.ds(i * 16, 16)
                    dst8[j, 0, iv] = mbig[pb][j, 1, iv]
                    cik8[j, 0, iv] = (mbig[pb][j, 2, iv] * NP
                                      + mbig[pb][j, 0, iv])

            pltpu.async_copy(xw2_ref.at[cik8.at[0, 0]], rows0, semg0)

            @pl.loop(0, BC // 2)
            def _pipe(p):
                for b in range(2):
                    cc = 2 * p + b
                    pltpu.make_async_copy(xw2_ref.at[cik8.at[cc, 0]],
                                          rows[b], semg[b]).wait()

                    @pl.when(cc + 1 < BC)
                    def _():
                        pltpu.async_copy(xw2_ref.at[cik8.at[cc + 1, 0]],
                                         rows[1 - b], semg[1 - b])

                    @pl.loop(0, CH // 16)
                    def _scale(g):
                        wv = ebig[pl.ds(cc * CH + g * 16, 16)]
                        for l in range(16):
                            ws = jnp.full((16,), wv[l], f32)
                            e = g * 16 + l
                            for j in range(D // 16):
                                jv = pl.ds(j * 16, 16)
                                rows[b][e, jv] = rows[b][e, jv] * ws

                    pltpu.sync_copy(rows[b], aggr.at[dst8.at[cc, 0]],
                                    add=True)

    plsc.subcore_barrier()
    pltpu.sync_copy(aggr.at[pl.ds(lo, RPS)],
                    agg_ref.at[cid, pl.ds(lo, RPS)])


def _p2(meta3, expv, xw2, mesh):
    f = functools.partial(
        pl.kernel,
        out_type=jax.ShapeDtypeStruct((2, NP, D), f32),
        mesh=mesh,
        scratch_types=[
            pltpu.VMEM((BC, 3, CH), i32),
            pltpu.VMEM((BC, 3, CH), i32),
            pltpu.VMEM((BC * CH,), f32),
            pltpu.VMEM((BC, 1, CH), i32),
            pltpu.VMEM((BC, 1, CH), i32),
            pltpu.VMEM((CH, D), f32),
            pltpu.VMEM((CH, D), f32),
            pltpu.VMEM_SHARED((NP, D), f32),
            pltpu.SemaphoreType.DMA,
            pltpu.SemaphoreType.DMA,
            pltpu.SemaphoreType.DMA,
            pltpu.SemaphoreType.DMA,
        ],
    )(_p2_body)
    return f(meta3, expv, xw2)


def _k4_body(a_ref, d_ref, b_ref, out_ref):
    ones = jnp.ones((2, 1), f32)
    s = lax.dot_general(d_ref[...], ones, (((0,), (0,)), ((), ())),
                        preferred_element_type=f32)
    rden = 1.0 / jnp.maximum(s, 1e-30)
    h = (a_ref[0] + a_ref[1]) * rden + b_ref[...]
    out_ref[...] = jnp.maximum(h, 0.0)


def _k4(agg, dpart, b):
    return pl.pallas_call(
        _k4_body,
        grid=(NB,),
        in_specs=[
            pl.BlockSpec((2, BN, D), lambda i: (0, i, 0)),
            pl.BlockSpec((2, BN), lambda i: (0, i)),
            pl.BlockSpec((1, D), lambda i: (0, 0)),
        ],
        out_specs=pl.BlockSpec((BN, D), lambda i: (i, 0)),
        out_shape=jax.ShapeDtypeStruct((NP, D), f32),
    )(agg, dpart, b.reshape(1, D))


def _layer(h, meta3, w, q, k, b, mesh):
    qT = q.reshape(1, D)
    kT = k.reshape(1, D)
    xw, sq, sk, gq, kv = _k1(h, w, qT, kT)
    expv, dpart = _p1(meta3, sq.reshape(R * NP), sk.reshape(R * NP),
                      gq.reshape(NP), kv.reshape(128), mesh)
    agg = _p2(meta3, expv, xw.reshape(R * NP, D), mesh)
    return _k4(agg, dpart, b)


def kernel(x, edge_index, edge_type, w0, q0, k0, b0, w1, q1, k1, b1):
    mesh = plsc.VectorSubcoreMesh(core_axis_name="c", subcore_axis_name="s")
    src = edge_index[0]
    dst = edge_index[1]
    pad = jnp.full((EP - E,), NP - 1, i32)
    srcp = jnp.concatenate([src, pad])
    dstp = jnp.concatenate([dst, pad])
    etp = jnp.concatenate([edge_type, jnp.zeros((EP - E,), i32)])
    meta3 = jnp.transpose(
        jnp.stack([srcp, dstp, etp]).reshape(3, EP // CH, CH), (1, 0, 2))
    xp = jnp.pad(x, ((0, NP - N), (0, 0)))
    h = _layer(xp, meta3, w0, q0, k0, b0, mesh)
    h = _layer(h, meta3, w1, q1, k1, b1, mesh)
    return h[:N]
